# baseline (speedup 1.0000x reference)
import functools

import jax
import jax.numpy as jnp
from jax import lax
from jax.experimental import pallas as pl
from jax.experimental.pallas import tpu as pltpu
from jax.experimental.pallas import tpu_sc as plsc

VOCAB = 1000000
EMBED = 64
EWIDE = 128
BEMBED = EMBED // 4
B = 16384
L = 50
N = B * L
LPAD = 56
EPAD = 128

LANES = 16
NC, NS = 2, 16
NW = NC * NS
B_PER_W = B // NW
CHUNK_B = 8
CHUNK = CHUNK_B * L
NCHUNK = B_PER_W // CHUNK_B
NPAIR = NCHUNK // 2
GD = [(0, 128), (128, 128), (256, 128), (384, 16)]


def _sc_body(ids_hbm, bids_hbm, table_hbm, btab_hbm, w_hbm, out_hbm,
             idx0, idx1, bidx0, bidx1, rows0, rows1, brows0, brows1, w_v,
             gsem0, gsem1, osem0, osem1):
    c = lax.axis_index("c")
    s = lax.axis_index("s")
    wid = s * NC + c
    flat0 = wid * B_PER_W * L
    b0 = wid * B_PER_W

    pltpu.sync_copy(w_hbm, w_v)
    wvec = w_v[:]

    idx = (idx0, idx1)
    bidx = (bidx0, bidx1)
    rows = (rows0, rows1)
    brows = (brows0, brows1)
    gsem = (gsem0, gsem1)
    osem = (osem0, osem1)

    def fire(n, b):
        base = flat0 + n * CHUNK
        pltpu.sync_copy(ids_hbm.at[pl.ds(base, CHUNK)], idx[b])
        pltpu.sync_copy(bids_hbm.at[pl.ds(base, CHUNK)], bidx[b])
        for off, cnt in GD:
            pltpu.async_copy(table_hbm.at[idx[b].at[pl.ds(off, cnt)]],
                             rows[b].at[pl.ds(off, cnt)], gsem[b])
            pltpu.async_copy(btab_hbm.at[bidx[b].at[pl.ds(off, cnt)]],
                             brows[b].at[pl.ds(off, cnt)], gsem[b])

    def drain_gather(b):
        for off, cnt in GD:
            pltpu.make_async_copy(table_hbm.at[idx[b].at[pl.ds(off, cnt)]],
                                  rows[b].at[pl.ds(off, cnt)],
                                  gsem[b]).wait()
            pltpu.make_async_copy(btab_hbm.at[bidx[b].at[pl.ds(off, cnt)]],
                                  brows[b].at[pl.ds(off, cnt)],
                                  gsem[b]).wait()

    def wait_out(b):
        for k in range(CHUNK_B):
            pltpu.make_async_copy(
                rows[b].at[pl.ds(k * L, L), pl.ds(0, EMBED)],
                out_hbm.at[b0 + k, pl.ds(0, L), pl.ds(0, EMBED)],
                osem[b]).wait()

    def compute_and_emit(n, b):
        drain_gather(b)

        def add_body(i, acc):
            bv = brows[b][i, :]
            plsc.addupdate(rows[b].at[i, pl.ds(0, BEMBED)], wvec * bv)
            return acc

        lax.fori_loop(0, CHUNK, add_body, 0, unroll=8)
        bb = b0 + n * CHUNK_B
        for k in range(CHUNK_B):
            pltpu.async_copy(
                rows[b].at[pl.ds(k * L, L), pl.ds(0, EMBED)],
                out_hbm.at[bb + k, pl.ds(0, L), pl.ds(0, EMBED)],
                osem[b])

    fire(0, 0)

    def pair_body(pg, carry):
        n0 = 2 * pg
        @pl.when(pg >= 1)
        def _():
            wait_out(1)

        fire(n0 + 1, 1)
        compute_and_emit(n0, 0)

        @pl.when(pg < NPAIR - 1)
        def _():
            wait_out(0)
            fire(n0 + 2, 0)

        compute_and_emit(n0 + 1, 1)
        return carry

    lax.fori_loop(0, NPAIR, pair_body, 0)
    wait_out(0)
    wait_out(1)


@functools.partial(jax.jit, donate_argnums=())
def kernel(input_ids, business_mask, table, business_table, context_weight):
    ids = input_ids.reshape(N)
    bids = business_mask.reshape(N)
    tablew = jnp.pad(table, ((0, 0), (0, EWIDE - EMBED)))
    wvec = jnp.broadcast_to(context_weight.astype(jnp.float32), (LANES,))

    run = pl.kernel(
        _sc_body,
        out_type=jax.ShapeDtypeStruct((B, LPAD, EPAD), jnp.float32),
        mesh=plsc.VectorSubcoreMesh(core_axis_name="c", subcore_axis_name="s"),
        scratch_types=[
            pltpu.VMEM((CHUNK,), jnp.int32),
            pltpu.VMEM((CHUNK,), jnp.int32),
            pltpu.VMEM((CHUNK,), jnp.int32),
            pltpu.VMEM((CHUNK,), jnp.int32),
            pltpu.VMEM((CHUNK, EWIDE), jnp.float32),
            pltpu.VMEM((CHUNK, EWIDE), jnp.float32),
            pltpu.VMEM((CHUNK, BEMBED), jnp.float32),
            pltpu.VMEM((CHUNK, BEMBED), jnp.float32),
            pltpu.VMEM((LANES,), jnp.float32),
            pltpu.SemaphoreType.DMA,
            pltpu.SemaphoreType.DMA,
            pltpu.SemaphoreType.DMA,
            pltpu.SemaphoreType.DMA,
        ],
        compiler_params=pltpu.CompilerParams(use_tc_tiling_on_sc=False),
    )
    out = run(ids, bids, tablew, business_table, wvec)
    return out[:, :L, :EMBED]


# speedup vs baseline: 1.0084x; 1.0084x over previous
import functools

import jax
import jax.numpy as jnp
from jax import lax
from jax.experimental import pallas as pl
from jax.experimental.pallas import tpu as pltpu
from jax.experimental.pallas import tpu_sc as plsc

EMBED = 64
BEMBED = EMBED // 4
B = 16384
L = 50
N = B * L
LPAD = 56
EPAD = 128

LANES = 16
NC, NS = 2, 16
NW = NC * NS
B_PER_W = B // NW
CHUNK_B = 8
CHUNK = CHUNK_B * L
NCHUNK = B_PER_W // CHUNK_B
NPAIR = NCHUNK // 2
GD = [(0, 128), (128, 128), (256, 128), (384, 16)]


def _sc_body(ids_hbm, bids_hbm, table_hbm, btab_hbm, w_hbm, out_hbm,
             idx0, idx1, bidx0, bidx1, rows0, rows1, brows0, brows1, w_v,
             gsem0, gsem1, osem0, osem1):
    c = lax.axis_index("c")
    s = lax.axis_index("s")
    wid = s * NC + c
    flat0 = wid * B_PER_W * L
    b0 = wid * B_PER_W

    pltpu.sync_copy(w_hbm, w_v)
    wvec = w_v[:]

    idx = (idx0, idx1)
    bidx = (bidx0, bidx1)
    rows = (rows0, rows1)
    brows = (brows0, brows1)
    gsem = (gsem0, gsem1)
    osem = (osem0, osem1)

    def fire(n, b):
        base = flat0 + n * CHUNK
        pltpu.sync_copy(ids_hbm.at[pl.ds(base, CHUNK)], idx[b])
        pltpu.sync_copy(bids_hbm.at[pl.ds(base, CHUNK)], bidx[b])
        for off, cnt in GD:
            pltpu.async_copy(table_hbm.at[idx[b].at[pl.ds(off, cnt)]],
                             rows[b].at[pl.ds(off, cnt)], gsem[b])
            pltpu.async_copy(btab_hbm.at[bidx[b].at[pl.ds(off, cnt)]],
                             brows[b].at[pl.ds(off, cnt)], gsem[b])

    def drain_gather(b):
        for off, cnt in GD:
            pltpu.make_async_copy(table_hbm.at[idx[b].at[pl.ds(off, cnt)]],
                                  rows[b].at[pl.ds(off, cnt)],
                                  gsem[b]).wait()
            pltpu.make_async_copy(btab_hbm.at[bidx[b].at[pl.ds(off, cnt)]],
                                  brows[b].at[pl.ds(off, cnt)],
                                  gsem[b]).wait()

    def wait_out(b):
        for k in range(CHUNK_B):
            pltpu.make_async_copy(
                rows[b].at[pl.ds(k * L, L)],
                out_hbm.at[b0 + k, pl.ds(0, L), pl.ds(0, EMBED)],
                osem[b]).wait()

    def compute_and_emit(n, b):
        drain_gather(b)

        def add_body(i, acc):
            bv = brows[b][i, :]
            plsc.addupdate(rows[b].at[i, pl.ds(0, BEMBED)], wvec * bv)
            return acc

        lax.fori_loop(0, CHUNK, add_body, 0, unroll=8)
        bb = b0 + n * CHUNK_B
        for k in range(CHUNK_B):
            pltpu.async_copy(
                rows[b].at[pl.ds(k * L, L)],
                out_hbm.at[bb + k, pl.ds(0, L), pl.ds(0, EMBED)],
                osem[b])

    fire(0, 0)

    def pair_body(pg, carry):
        n0 = 2 * pg
        @pl.when(pg >= 1)
        def _():
            wait_out(1)

        fire(n0 + 1, 1)
        compute_and_emit(n0, 0)

        @pl.when(pg < NPAIR - 1)
        def _():
            wait_out(0)
            fire(n0 + 2, 0)

        compute_and_emit(n0 + 1, 1)
        return carry

    lax.fori_loop(0, NPAIR, pair_body, 0)
    wait_out(0)
    wait_out(1)


@functools.partial(jax.jit, donate_argnums=())
def kernel(input_ids, business_mask, table, business_table, context_weight):
    ids = input_ids.reshape(N)
    bids = business_mask.reshape(N)
    wvec = jnp.broadcast_to(context_weight.astype(jnp.float32), (LANES,))

    run = pl.kernel(
        _sc_body,
        out_type=jax.ShapeDtypeStruct((B, LPAD, EPAD), jnp.float32),
        mesh=plsc.VectorSubcoreMesh(core_axis_name="c", subcore_axis_name="s"),
        scratch_types=[
            pltpu.VMEM((CHUNK,), jnp.int32),
            pltpu.VMEM((CHUNK,), jnp.int32),
            pltpu.VMEM((CHUNK,), jnp.int32),
            pltpu.VMEM((CHUNK,), jnp.int32),
            pltpu.VMEM((CHUNK, EMBED), jnp.float32),
            pltpu.VMEM((CHUNK, EMBED), jnp.float32),
            pltpu.VMEM((CHUNK, BEMBED), jnp.float32),
            pltpu.VMEM((CHUNK, BEMBED), jnp.float32),
            pltpu.VMEM((LANES,), jnp.float32),
            pltpu.SemaphoreType.DMA,
            pltpu.SemaphoreType.DMA,
            pltpu.SemaphoreType.DMA,
            pltpu.SemaphoreType.DMA,
        ],
        compiler_params=pltpu.CompilerParams(use_tc_tiling_on_sc=False),
    )
    out = run(ids, bids, table, business_table, wvec)
    return out[:, :L, :EMBED]


# speedup vs baseline: 1.0349x; 1.0263x over previous
import functools

import jax
import jax.numpy as jnp
from jax import lax
from jax.experimental import pallas as pl
from jax.experimental.pallas import tpu as pltpu
from jax.experimental.pallas import tpu_sc as plsc

EMBED = 64
BEMBED = EMBED // 4
B = 16384
L = 50
N = B * L
LPAD = 56
EPAD = 128

LANES = 16
NC, NS = 2, 16
NW = NC * NS
B_PER_W = B // NW
CHUNK_B = 8
CHUNK = CHUNK_B * L
NCHUNK = B_PER_W // CHUNK_B
NPAIR = NCHUNK // 2
GD = [(0, 128), (128, 128), (256, 128), (384, 16)]


def _sc_body(ids_hbm, bids_hbm, table_hbm, btab_hbm, w_hbm, out_hbm,
             idx0, idx1, bidx0, bidx1, rows0, rows1, brows0, brows1, w_v,
             gsem0, gsem1, osem0, osem1, isem0, isem1):
    c = lax.axis_index("c")
    s = lax.axis_index("s")
    wid = s * NC + c
    flat0 = wid * B_PER_W * L
    b0 = wid * B_PER_W

    pltpu.sync_copy(w_hbm, w_v)
    wvec = w_v[:]

    idx = (idx0, idx1)
    bidx = (bidx0, bidx1)
    rows = (rows0, rows1)
    brows = (brows0, brows1)
    gsem = (gsem0, gsem1)
    osem = (osem0, osem1)
    isem = (isem0, isem1)

    def stage(n, b):
        base = flat0 + n * CHUNK
        pltpu.async_copy(ids_hbm.at[pl.ds(base, CHUNK)], idx[b], isem[b])
        pltpu.async_copy(bids_hbm.at[pl.ds(base, CHUNK)], bidx[b], isem[b])

    def fire(n, b):
        base = flat0 + n * CHUNK
        pltpu.make_async_copy(ids_hbm.at[pl.ds(base, CHUNK)], idx[b],
                              isem[b]).wait()
        pltpu.make_async_copy(bids_hbm.at[pl.ds(base, CHUNK)], bidx[b],
                              isem[b]).wait()
        for off, cnt in GD:
            pltpu.async_copy(table_hbm.at[idx[b].at[pl.ds(off, cnt)]],
                             rows[b].at[pl.ds(off, cnt)], gsem[b])
            pltpu.async_copy(btab_hbm.at[bidx[b].at[pl.ds(off, cnt)]],
                             brows[b].at[pl.ds(off, cnt)], gsem[b])

    def drain_gather(b):
        for off, cnt in GD:
            pltpu.make_async_copy(table_hbm.at[idx[b].at[pl.ds(off, cnt)]],
                                  rows[b].at[pl.ds(off, cnt)],
                                  gsem[b]).wait()
            pltpu.make_async_copy(btab_hbm.at[bidx[b].at[pl.ds(off, cnt)]],
                                  brows[b].at[pl.ds(off, cnt)],
                                  gsem[b]).wait()

    def wait_out(b):
        for k in range(CHUNK_B):
            pltpu.make_async_copy(
                rows[b].at[pl.ds(k * L, L)],
                out_hbm.at[b0 + k, pl.ds(0, L), pl.ds(0, EMBED)],
                osem[b]).wait()

    def compute_and_emit(n, b):
        drain_gather(b)

        @pl.when(n + 2 < NCHUNK)
        def _():
            stage(n + 2, b)

        def add_body(i, acc):
            bv = brows[b][i, :]
            plsc.addupdate(rows[b].at[i, pl.ds(0, BEMBED)], wvec * bv)
            return acc

        lax.fori_loop(0, CHUNK, add_body, 0, unroll=8)
        bb = b0 + n * CHUNK_B
        for k in range(CHUNK_B):
            pltpu.async_copy(
                rows[b].at[pl.ds(k * L, L)],
                out_hbm.at[bb + k, pl.ds(0, L), pl.ds(0, EMBED)],
                osem[b])

    stage(0, 0)
    stage(1, 1)
    fire(0, 0)

    def pair_body(pg, carry):
        n0 = 2 * pg
        @pl.when(pg >= 1)
        def _():
            wait_out(1)

        fire(n0 + 1, 1)
        compute_and_emit(n0, 0)

        @pl.when(pg < NPAIR - 1)
        def _():
            wait_out(0)
            fire(n0 + 2, 0)

        compute_and_emit(n0 + 1, 1)
        return carry

    lax.fori_loop(0, NPAIR, pair_body, 0)
    wait_out(0)
    wait_out(1)


@functools.partial(jax.jit, donate_argnums=())
def kernel(input_ids, business_mask, table, business_table, context_weight):
    ids = input_ids.reshape(N)
    bids = business_mask.reshape(N)
    wvec = jnp.broadcast_to(context_weight.astype(jnp.float32), (LANES,))

    run = pl.kernel(
        _sc_body,
        out_type=jax.ShapeDtypeStruct((B, LPAD, EPAD), jnp.float32),
        mesh=plsc.VectorSubcoreMesh(core_axis_name="c", subcore_axis_name="s"),
        scratch_types=[
            pltpu.VMEM((CHUNK,), jnp.int32),
            pltpu.VMEM((CHUNK,), jnp.int32),
            pltpu.VMEM((CHUNK,), jnp.int32),
            pltpu.VMEM((CHUNK,), jnp.int32),
            pltpu.VMEM((CHUNK, EMBED), jnp.float32),
            pltpu.VMEM((CHUNK, EMBED), jnp.float32),
            pltpu.VMEM((CHUNK, BEMBED), jnp.float32),
            pltpu.VMEM((CHUNK, BEMBED), jnp.float32),
            pltpu.VMEM((LANES,), jnp.float32),
            pltpu.SemaphoreType.DMA,
            pltpu.SemaphoreType.DMA,
            pltpu.SemaphoreType.DMA,
            pltpu.SemaphoreType.DMA,
            pltpu.SemaphoreType.DMA,
            pltpu.SemaphoreType.DMA,
        ],
        compiler_params=pltpu.CompilerParams(use_tc_tiling_on_sc=False),
    )
    out = run(ids, bids, table, business_table, wvec)
    return out[:, :L, :EMBED]
